# baseline (speedup 1.0000x reference)
import functools

import jax
import jax.numpy as jnp
from jax import lax
from jax.experimental import pallas as pl
from jax.experimental.pallas import tpu as pltpu
from jax.experimental.pallas import tpu_sc as plsc

NSP = 9
EPS = 1e-10
WS = 7
OFF = WS // 2
LANES = 16
NUM_TEC = 16
H_, W_, K_ = 56, 56, 49
RB = 8
NB = H_ // RB
HALO = RB + 2 * OFF
WK = W_ * K_
WKP = WK + 8
WP = 128
SLAB = K_ * RB * WP
PIXROW = W_ * NSP
NCHUNK = (WK + LANES - 1) // LANES


def _make_sc_body(bfix):
    def body(sims_hbm, sinds_hbm, tbl_hbm, out_hbm,
             sims_v, sinds_v, ph_v, tbl_v, mb_v):
        wid = lax.axis_index("c") * NUM_TEC + lax.axis_index("s")
        hb = wid // 4
        grp = wid - hb * 4
        r0 = hb * RB
        lo = jnp.clip(r0 - OFF, 0, H_ - HALO)
        s_lo = grp * 2
        s_hi = jnp.where(grp == 3, NSP, grp * 2 + 2)

        def work(_, __):
            inoff = pl.multiple_of((bfix * H_ + lo) * PIXROW, 8)
            pltpu.sync_copy(tbl_hbm, tbl_v)
            pltpu.sync_copy(sims_hbm.at[pl.ds(inoff, HALO * PIXROW)], sims_v)
            pltpu.sync_copy(sinds_hbm.at[pl.ds(inoff, HALO * PIXROW)],
                            sinds_v)

            zero = jnp.zeros((LANES,), jnp.float32)

            def zbody(i, _):
                ph_v[pl.ds(i * LANES, LANES)] = zero
                return 0
            lax.fori_loop(0, (HALO * WK) // LANES, zbody, 0)

            iota = lax.broadcasted_iota(jnp.int32, (LANES,), 0)
            pix9 = iota * NSP
            i49 = iota * K_

            def sbody(ci, _):
                for sp in range(NSP):
                    idxv = pix9 + (ci * (LANES * NSP) + sp)
                    sindv = plsc.load_gather(sinds_v, [idxv])
                    simsv = plsc.load_gather(sims_v, [idxv])
                    pidx = i49 + ci * (LANES * K_) + sindv
                    plsc.addupdate_scatter(ph_v, [pidx], simsv)
                return 0
            lax.fori_loop(0, (HALO * W_) // LANES, sbody, 0)

            def s_body(s, _):
                def cbody(c, _):
                    base = c * LANES
                    dhv = tbl_v[pl.ds(base, LANES)]
                    col49v = tbl_v[pl.ds(WKP + base, LANES)]
                    w9v = tbl_v[pl.ds(2 * WKP + base, LANES)]
                    didxv = tbl_v[pl.ds(3 * WKP + base, LANES)]
                    sidloc = w9v + s
                    for r in range(RB):
                        h = r0 + r
                        rv = jnp.clip(dhv + h, 0, H_ - 1)
                        geo = (rv - lo) * WK + col49v
                        sidv = plsc.load_gather(
                            sinds_v, [sidloc + (h - lo) * PIXROW])
                        val = plsc.load_gather(ph_v, [geo + sidv])
                        plsc.store_scatter(mb_v, [didxv + r * WP], val)
                    return 0
                lax.fori_loop(0, NCHUNK, cbody, 0)
                slaboff = (hb * NSP + s) * SLAB
                pltpu.sync_copy(mb_v.at[pl.ds(0, SLAB)],
                                out_hbm.at[pl.ds(pl.multiple_of(slaboff, 8),
                                                 SLAB)])
                return 0
            lax.fori_loop(s_lo, s_hi, s_body, 0)
            return 0
        lax.fori_loop(0, jnp.where(wid < 4 * NB, 1, 0), work, 0)
    return body


def _dense_body(c_ref, attn_ref, match_ref, out_ref):
    HD = attn_ref.shape[1]
    c = c_ref[0, 0]
    for hd in range(HD):
        e3 = jnp.exp(attn_ref[0, hd] - c)
        for s in range(NSP):
            m3 = match_ref[0, s][:, :, 0:W_]
            em3 = e3 * m3
            den = jnp.sum(em3, axis=0)
            out_ref[0, hd, s] = em3 * (1.0 / (EPS + den))[None]


def _dense_body2(c_ref, attn_ref, match_ref, prev_ref, out_ref):
    _dense_body(c_ref, attn_ref, match_ref, out_ref)


@jax.jit
def kernel(attn, sims, sinds):
    B, HD, H, W, K = attn.shape
    c = jnp.max(attn).reshape(1, 1)

    f = jnp.arange(WK, dtype=jnp.int32)
    wcol = f // K
    kk = f - wcol * K
    dh = kk // WS - OFF
    dw = kk - (kk // WS) * WS - OFF
    col49 = jnp.clip(wcol + dw, 0, W - 1) * K
    didx = kk * (RB * WP) + wcol
    pad = jnp.zeros((WKP - WK,), jnp.int32)
    dpad = jnp.full((WKP - WK,), SLAB, jnp.int32)
    tbl = jnp.concatenate([dh, pad, col49, pad, wcol * NSP, pad,
                           didx, dpad]).astype(jnp.int32)

    sims_flat = sims.reshape(B * H * W * NSP)
    sinds_flat = sinds.reshape(B * H * W * NSP)

    mesh = plsc.VectorSubcoreMesh(core_axis_name="c", subcore_axis_name="s")
    scratch = [
        pltpu.VMEM((HALO * W * NSP,), jnp.float32),
        pltpu.VMEM((HALO * W * NSP,), jnp.int32),
        pltpu.VMEM((HALO * WK,), jnp.float32),
        pltpu.VMEM((4 * WKP,), jnp.int32),
        pltpu.VMEM((SLAB + RB * WP,), jnp.float32),
    ]
    match_b = [
        pl.kernel(
            _make_sc_body(b),
            out_type=jax.ShapeDtypeStruct((NB * NSP * SLAB,), jnp.float32),
            mesh=mesh,
            compiler_params=pltpu.CompilerParams(needs_layout_passes=False),
            scratch_types=scratch,
        )(sims_flat, sinds_flat, tbl).reshape(NB, NSP, K, RB, WP)
        for b in range(B)
    ]

    attn_t = jnp.transpose(attn, (0, 1, 4, 2, 3))
    out_shape = jax.ShapeDtypeStruct((B, HD, NSP, K, H, W), jnp.float32)

    def dense_call(b, body, extra_specs, aliases):
        return pl.pallas_call(
            body,
            grid=(NB,),
            in_specs=[
                pl.BlockSpec((1, 1), lambda hb: (0, 0)),
                pl.BlockSpec((1, HD, K, RB, W), lambda hb: (b, 0, 0, hb, 0)),
                pl.BlockSpec((1, NSP, K, RB, WP), lambda hb: (hb, 0, 0, 0, 0)),
            ] + extra_specs,
            out_specs=pl.BlockSpec((1, HD, NSP, K, RB, W),
                                   lambda hb: (b, 0, 0, 0, hb, 0)),
            out_shape=out_shape,
            input_output_aliases=aliases,
            compiler_params=pltpu.CompilerParams(
                dimension_semantics=("parallel",)),
        )

    out0 = dense_call(0, _dense_body, [], {})(c, attn_t, match_b[0])
    out_phys = dense_call(
        1, _dense_body2, [pl.BlockSpec(memory_space=pl.ANY)], {3: 0},
    )(c, attn_t, match_b[1], out0)
    return jnp.transpose(out_phys, (0, 1, 2, 4, 5, 3))
